# Initial kernel scaffold; baseline (speedup 1.0000x reference)
import jax
import jax.numpy as jnp
from jax.experimental import pallas as pl


def kernel(F0, vertices0, k, W1, b1, W2, b2):
    raise NotImplementedError("write your pallas kernel here")


# speedup vs baseline: 1.1743x; 1.1743x over previous
import functools

import jax
import jax.numpy as jnp
from jax import lax
from jax.experimental import pallas as pl
from jax.experimental.pallas import tpu as pltpu
from jax.experimental.pallas import tpu_sc as plsc

B, N, S, D = 2048, 40, 20, 512
H = 256
BB = 128
VP = 16
NW = 32
RPW = B * S // NW
CH = 64
NCH = RPW // CH


def _tc_body(f_ref, v_ref, w1_ref, b1_ref, w2_ref, b2_ref, score_ref,
             fidx_ref, vnew_ref):
    X = f_ref[...]
    Xf = X.reshape(BB * N, D)
    h = jnp.dot(Xf, w1_ref[...], preferred_element_type=jnp.float32)
    h = h + b1_ref[...]
    h = jnp.where(h >= 0, h, 0.2 * h)
    sc = jnp.dot(h, w2_ref[...], preferred_element_type=jnp.float32)
    sc = sc + b2_ref[...]
    sc3 = sc.reshape(BB, N, N)
    score_ref[...] = sc3

    pooled = jnp.sum(X, axis=1) * (1.0 / N)
    hp = jnp.dot(pooled, w1_ref[...], preferred_element_type=jnp.float32)
    hp = hp + b1_ref[...]
    hp = jnp.where(hp >= 0, hp, 0.2 * hp)
    pli = jnp.dot(hp, w2_ref[...], preferred_element_type=jnp.float32)
    pli = pli + b2_ref[...]

    iota = lax.broadcasted_iota(jnp.int32, (BB, N), 1)
    pmax = jnp.max(pli, axis=-1, keepdims=True)
    pred = jnp.min(jnp.where(pli == pmax, iota, N), axis=-1, keepdims=True)
    onehot = (iota == pred).astype(jnp.float32)
    score_f = jnp.sum(sc3 * onehot[:, None, :], axis=-1)

    w = score_f
    cols = []
    neg = jnp.float32(-jnp.inf)
    for _ in range(S):
        m = jnp.max(w, axis=-1, keepdims=True)
        i = jnp.min(jnp.where(w == m, iota, N), axis=-1, keepdims=True)
        cols.append(i)
        w = jnp.where(iota == i, neg, w)
    idx = jnp.concatenate(cols, axis=1)
    row0 = pl.program_id(0) * BB
    rows = row0 + lax.broadcasted_iota(jnp.int32, (BB, S), 0)
    fidx_ref[...] = rows * N + idx

    iota_n3 = lax.broadcasted_iota(jnp.int32, (BB, S, N), 2)
    oh3 = (idx[:, :, None] == iota_n3).astype(jnp.float32)
    vnew_ref[...] = lax.dot_general(
        oh3, v_ref[...], (((2,), (1,)), ((0,), (0,))),
        preferred_element_type=jnp.float32)


def _tc_score(F0, vertices0, W1, b1, W2, b2):
    grid = (B // BB,)
    return pl.pallas_call(
        _tc_body,
        grid=grid,
        in_specs=[
            pl.BlockSpec((BB, N, D), lambda i: (i, 0, 0)),
            pl.BlockSpec((BB, N, 3), lambda i: (i, 0, 0)),
            pl.BlockSpec((D, H), lambda i: (0, 0)),
            pl.BlockSpec((1, H), lambda i: (0, 0)),
            pl.BlockSpec((H, N), lambda i: (0, 0)),
            pl.BlockSpec((1, N), lambda i: (0, 0)),
        ],
        out_specs=[
            pl.BlockSpec((BB, N, N), lambda i: (i, 0, 0)),
            pl.BlockSpec((BB, S), lambda i: (i, 0)),
            pl.BlockSpec((BB, S, 3), lambda i: (i, 0, 0)),
        ],
        out_shape=[
            jax.ShapeDtypeStruct((B, N, N), jnp.float32),
            jax.ShapeDtypeStruct((B, S), jnp.int32),
            jax.ShapeDtypeStruct((B, S, 3), jnp.float32),
        ],
        compiler_params=pltpu.CompilerParams(
            dimension_semantics=("arbitrary",),
        ),
    )(F0, vertices0, W1, b1, W2, b2)


def _sc_gather(Fflat, idx3d):
    mesh = plsc.VectorSubcoreMesh(core_axis_name="c", subcore_axis_name="s")

    @functools.partial(
        pl.kernel,
        mesh=mesh,
        out_type=jax.ShapeDtypeStruct((B * S, D), jnp.float32),
        scratch_types=[
            pltpu.VMEM((NCH, CH), jnp.int32),
            pltpu.VMEM((2, CH, D), jnp.float32),
            pltpu.SemaphoreType.DMA,
            pltpu.SemaphoreType.DMA,
        ],
    )
    def k(f_hbm, idx_hbm, outf_hbm, idx_v, fbuf, gsem, osem):
        wid = lax.axis_index("s") * 2 + lax.axis_index("c")
        pltpu.sync_copy(idx_hbm.at[wid], idx_v)
        base = wid * RPW
        cp = pltpu.async_copy(f_hbm.at[idx_v.at[0]], fbuf.at[0], gsem)
        for c in range(NCH):
            cp.wait()
            if c + 1 < NCH:
                cp = pltpu.async_copy(f_hbm.at[idx_v.at[c + 1]],
                                      fbuf.at[(c + 1) % 2], gsem)
            if c > 0:
                wr.wait()
            wr = pltpu.async_copy(fbuf.at[c % 2],
                                  outf_hbm.at[pl.ds(base + c * CH, CH)], osem)
        wr.wait()

    return k(Fflat, idx3d)


def kernel(F0, vertices0, k, W1, b1, W2, b2):
    score, fidx, vnew = _tc_score(F0, vertices0, W1, b1.reshape(1, H),
                                  W2, b2.reshape(1, N))
    Ff = _sc_gather(F0.reshape(B * N, D), fidx.reshape(NW, NCH, CH))
    return (Ff.reshape(B, S, D), score, vnew)


# speedup vs baseline: 1.9702x; 1.6777x over previous
import functools

import jax
import jax.numpy as jnp
from jax import lax
from jax.experimental import pallas as pl
from jax.experimental.pallas import tpu as pltpu
from jax.experimental.pallas import tpu_sc as plsc

B, N, S, D = 2048, 40, 20, 512
H = 256
BB = 128
NW = 32
CW = B // NW


def _tc_body(f_ref, v_ref, w1_ref, b1_ref, w2_ref, b2_ref, score_ref,
             fidx_ref, vnew_ref):
    X = f_ref[...]
    Xf = X.reshape(BB * N, D)
    h = jnp.dot(Xf, w1_ref[...], preferred_element_type=jnp.float32)
    h = h + b1_ref[...]
    h = jnp.where(h >= 0, h, 0.2 * h)
    sc = jnp.dot(h, w2_ref[...], preferred_element_type=jnp.float32)
    sc = sc + b2_ref[...]
    sc3 = sc.reshape(BB, N, N)
    score_ref[...] = sc3

    pooled = jnp.sum(X, axis=1) * (1.0 / N)
    hp = jnp.dot(pooled, w1_ref[...], preferred_element_type=jnp.float32)
    hp = hp + b1_ref[...]
    hp = jnp.where(hp >= 0, hp, 0.2 * hp)
    pli = jnp.dot(hp, w2_ref[...], preferred_element_type=jnp.float32)
    pli = pli + b2_ref[...]

    iota = lax.broadcasted_iota(jnp.int32, (BB, N), 1)
    pmax = jnp.max(pli, axis=-1, keepdims=True)
    pred = jnp.min(jnp.where(pli == pmax, iota, N), axis=-1, keepdims=True)
    onehot = (iota == pred).astype(jnp.float32)
    score_f = jnp.sum(sc3 * onehot[:, None, :], axis=-1)

    w = score_f
    cols = []
    neg = jnp.float32(-jnp.inf)
    for _ in range(S):
        m = jnp.max(w, axis=-1, keepdims=True)
        i = jnp.min(jnp.where(w == m, iota, N), axis=-1, keepdims=True)
        cols.append(i)
        w = jnp.where(iota == i, neg, w)
    idx = jnp.concatenate(cols, axis=1)
    idx_t = idx.T
    row0 = pl.program_id(0) * BB
    rows = row0 + lax.broadcasted_iota(jnp.int32, (S, BB), 1)
    fidx_ref[...] = rows * N + idx_t

    iota_n3 = lax.broadcasted_iota(jnp.int32, (BB, S, N), 2)
    oh3 = (idx[:, :, None] == iota_n3).astype(jnp.float32)
    vt = v_ref[...]
    for c in range(3):
        sel = jnp.sum(oh3 * vt[c][:, None, :], axis=-1)
        vnew_ref[c] = sel.T


def _tc_score(F0, vertices0, W1, b1, W2, b2):
    grid = (B // BB,)
    return pl.pallas_call(
        _tc_body,
        grid=grid,
        in_specs=[
            pl.BlockSpec((BB, N, D), lambda i: (i, 0, 0)),
            pl.BlockSpec((3, BB, N), lambda i: (0, i, 0)),
            pl.BlockSpec((D, H), lambda i: (0, 0)),
            pl.BlockSpec((1, H), lambda i: (0, 0)),
            pl.BlockSpec((H, N), lambda i: (0, 0)),
            pl.BlockSpec((1, N), lambda i: (0, 0)),
        ],
        out_specs=[
            pl.BlockSpec((BB, N, N), lambda i: (i, 0, 0)),
            pl.BlockSpec((S, BB), lambda i: (0, i)),
            pl.BlockSpec((3, S, BB), lambda i: (0, 0, i)),
        ],
        out_shape=[
            jax.ShapeDtypeStruct((B, N, N), jnp.float32),
            jax.ShapeDtypeStruct((S, B), jnp.int32),
            jax.ShapeDtypeStruct((3, S, B), jnp.float32),
        ],
        compiler_params=pltpu.CompilerParams(
            dimension_semantics=("arbitrary",),
        ),
    )(F0, vertices0, W1, b1, W2, b2)


def _sc_gather(Fflat, idxt):
    mesh = plsc.VectorSubcoreMesh(core_axis_name="c", subcore_axis_name="s")

    @functools.partial(
        pl.kernel,
        mesh=mesh,
        out_type=jax.ShapeDtypeStruct((S * B, D), jnp.float32),
        scratch_types=[
            pltpu.VMEM((S, 2 * CW), jnp.int32),
            pltpu.VMEM((2, CW, D), jnp.float32),
            pltpu.SemaphoreType.DMA,
            pltpu.SemaphoreType.DMA,
        ],
    )
    def k(f_hbm, idx_hbm, outf_hbm, idx_v, fbuf, gsem, osem):
        wid = lax.axis_index("s") * 2 + lax.axis_index("c")
        pltpu.sync_copy(idx_hbm.at[:, pl.ds((wid // 2) * 2 * CW, 2 * CW)],
                        idx_v)
        half = (wid % 2) * CW
        cp = pltpu.async_copy(f_hbm.at[idx_v.at[0, pl.ds(half, CW)]],
                              fbuf.at[0], gsem)
        for s in range(S):
            cp.wait()
            if s + 1 < S:
                cp = pltpu.async_copy(
                    f_hbm.at[idx_v.at[s + 1, pl.ds(half, CW)]],
                    fbuf.at[(s + 1) % 2], gsem)
            if s > 0:
                wr.wait()
            wr = pltpu.async_copy(
                fbuf.at[s % 2],
                outf_hbm.at[pl.ds(s * B + wid * CW, CW)], osem)
        wr.wait()

    return k(Fflat, idxt)


def kernel(F0, vertices0, k, W1, b1, W2, b2):
    score, fidx_t, vnew_t = _tc_score(
        F0, jnp.transpose(vertices0, (2, 0, 1)), W1, b1.reshape(1, H),
        W2, b2.reshape(1, N))
    Ff = _sc_gather(F0.reshape(B * N, D), fidx_t)
    F_new = jnp.transpose(Ff.reshape(S, B, D), (1, 0, 2))
    vertices_new = jnp.transpose(vnew_t, (2, 1, 0))
    return (F_new, score, vertices_new)


# speedup vs baseline: 3.1890x; 1.6186x over previous
import functools

import jax
import jax.numpy as jnp
from jax import lax
from jax.experimental import pallas as pl
from jax.experimental.pallas import tpu as pltpu
from jax.experimental.pallas import tpu_sc as plsc

B, N, S, D = 2048, 40, 20, 512
H = 256
BB = 128
NW = 32
CW = B // NW


def _tc_body(f_ref, v_ref, w1_ref, b1_ref, w2_ref, b2_ref, score_ref,
             fidx_ref, vnew_ref):
    X = f_ref[...]
    Xf = X.reshape(BB * N, D)
    h = jnp.dot(Xf, w1_ref[...], preferred_element_type=jnp.float32)
    h = h + b1_ref[...]
    h = jnp.where(h >= 0, h, 0.2 * h)
    sc = jnp.dot(h, w2_ref[...], preferred_element_type=jnp.float32)
    sc = sc + b2_ref[...]
    sc3 = sc.reshape(BB, N, N)
    score_ref[...] = sc3

    pooled = jnp.sum(X, axis=1) * (1.0 / N)
    hp = jnp.dot(pooled, w1_ref[...], preferred_element_type=jnp.float32)
    hp = hp + b1_ref[...]
    hp = jnp.where(hp >= 0, hp, 0.2 * hp)
    pli = jnp.dot(hp, w2_ref[...], preferred_element_type=jnp.float32)
    pli = pli + b2_ref[...]

    iota = lax.broadcasted_iota(jnp.int32, (BB, N), 1)
    pmax = jnp.max(pli, axis=-1, keepdims=True)
    pred = jnp.min(jnp.where(pli == pmax, iota, N), axis=-1, keepdims=True)
    onehot = (iota == pred).astype(jnp.float32)
    score_f = jnp.sum(sc3 * onehot[:, None, :], axis=-1)

    w = score_f.T
    vt = v_ref[...]
    iota_n = lax.broadcasted_iota(jnp.int32, (N, BB), 0)
    neg = jnp.float32(-jnp.inf)
    rows_i, vrows = [], [[], [], []]
    for _ in range(S):
        m = jnp.max(w, axis=0, keepdims=True)
        i = jnp.min(jnp.where(w == m, iota_n, N), axis=0, keepdims=True)
        rows_i.append(i)
        msk = (iota_n == i)
        w = jnp.where(msk, neg, w)
        mf = msk.astype(jnp.float32)
        for c in range(3):
            vrows[c].append(jnp.sum(mf * vt[c], axis=0, keepdims=True))
    idx_t = jnp.concatenate(rows_i, axis=0)
    row0 = pl.program_id(0) * BB
    rows = row0 + lax.broadcasted_iota(jnp.int32, (S, BB), 1)
    fidx_ref[...] = rows * N + idx_t
    for c in range(3):
        vnew_ref[c] = jnp.concatenate(vrows[c], axis=0)


def _tc_score(F0, vertices0, W1, b1, W2, b2):
    grid = (B // BB,)
    return pl.pallas_call(
        _tc_body,
        grid=grid,
        in_specs=[
            pl.BlockSpec((BB, N, D), lambda i: (i, 0, 0)),
            pl.BlockSpec((3, N, BB), lambda i: (0, 0, i)),
            pl.BlockSpec((D, H), lambda i: (0, 0)),
            pl.BlockSpec((1, H), lambda i: (0, 0)),
            pl.BlockSpec((H, N), lambda i: (0, 0)),
            pl.BlockSpec((1, N), lambda i: (0, 0)),
        ],
        out_specs=[
            pl.BlockSpec((BB, N, N), lambda i: (i, 0, 0)),
            pl.BlockSpec((S, BB), lambda i: (0, i)),
            pl.BlockSpec((3, S, BB), lambda i: (0, 0, i)),
        ],
        out_shape=[
            jax.ShapeDtypeStruct((B, N, N), jnp.float32),
            jax.ShapeDtypeStruct((S, B), jnp.int32),
            jax.ShapeDtypeStruct((3, S, B), jnp.float32),
        ],
        compiler_params=pltpu.CompilerParams(
            dimension_semantics=("arbitrary",),
        ),
    )(F0, vertices0, W1, b1, W2, b2)


def _sc_gather(Fflat, idxt):
    mesh = plsc.VectorSubcoreMesh(core_axis_name="c", subcore_axis_name="s")

    @functools.partial(
        pl.kernel,
        mesh=mesh,
        out_type=jax.ShapeDtypeStruct((S * B, D), jnp.float32),
        scratch_types=[
            pltpu.VMEM((S, 2 * CW), jnp.int32),
            pltpu.VMEM((2, CW, D), jnp.float32),
            pltpu.SemaphoreType.DMA,
            pltpu.SemaphoreType.DMA,
        ],
    )
    def k(f_hbm, idx_hbm, outf_hbm, idx_v, fbuf, gsem, osem):
        wid = lax.axis_index("s") * 2 + lax.axis_index("c")
        pltpu.sync_copy(idx_hbm.at[:, pl.ds((wid // 2) * 2 * CW, 2 * CW)],
                        idx_v)
        half = (wid % 2) * CW
        cp = pltpu.async_copy(f_hbm.at[idx_v.at[0, pl.ds(half, CW)]],
                              fbuf.at[0], gsem)
        for s in range(S):
            cp.wait()
            if s + 1 < S:
                cp = pltpu.async_copy(
                    f_hbm.at[idx_v.at[s + 1, pl.ds(half, CW)]],
                    fbuf.at[(s + 1) % 2], gsem)
            if s > 0:
                wr.wait()
            wr = pltpu.async_copy(
                fbuf.at[s % 2],
                outf_hbm.at[pl.ds(s * B + wid * CW, CW)], osem)
        wr.wait()

    return k(Fflat, idxt)


def kernel(F0, vertices0, k, W1, b1, W2, b2):
    score, fidx_t, vnew_t = _tc_score(
        F0, jnp.transpose(vertices0, (2, 1, 0)), W1, b1.reshape(1, H),
        W2, b2.reshape(1, N))
    Ff = _sc_gather(F0.reshape(B * N, D), fidx_t)
    F_new = jnp.transpose(Ff.reshape(S, B, D), (1, 0, 2))
    vertices_new = jnp.transpose(vnew_t, (2, 1, 0))
    return (F_new, score, vertices_new)


# speedup vs baseline: 3.2062x; 1.0054x over previous
import functools

import jax
import jax.numpy as jnp
from jax import lax
from jax.experimental import pallas as pl
from jax.experimental.pallas import tpu as pltpu
from jax.experimental.pallas import tpu_sc as plsc

B, N, S, D = 2048, 40, 20, 512
H = 256
BB = 128
NW = 32
CW = B // NW


def _tc_body(f_ref, v_ref, w1_ref, b1_ref, w2_ref, b2_ref, score_ref,
             fidx_ref, vnew_ref):
    X = f_ref[...]
    Xf = X.reshape(BB * N, D)
    h = jnp.dot(Xf, w1_ref[...], preferred_element_type=jnp.float32)
    h = h + b1_ref[...]
    h = jnp.where(h >= 0, h, 0.2 * h)
    sc = jnp.dot(h, w2_ref[...], preferred_element_type=jnp.float32)
    sc = sc + b2_ref[...]
    sc3 = sc.reshape(BB, N, N)
    score_ref[...] = sc3

    pooled = jnp.sum(X, axis=1) * (1.0 / N)
    hp = jnp.dot(pooled, w1_ref[...], preferred_element_type=jnp.float32)
    hp = hp + b1_ref[...]
    hp = jnp.where(hp >= 0, hp, 0.2 * hp)
    pli = jnp.dot(hp, w2_ref[...], preferred_element_type=jnp.float32)
    pli = pli + b2_ref[...]

    iota = lax.broadcasted_iota(jnp.int32, (BB, N), 1)
    pmax = jnp.max(pli, axis=-1, keepdims=True)
    pred = jnp.min(jnp.where(pli == pmax, iota, N), axis=-1, keepdims=True)
    onehot = (iota == pred).astype(jnp.float32)
    score_f = jnp.sum(sc3 * onehot[:, None, :], axis=-1)

    w = score_f.T
    vt = v_ref[...]
    iota_n = lax.broadcasted_iota(jnp.int32, (N, BB), 0)
    neg = jnp.float32(-jnp.inf)
    rows_i, vrows = [], [[], [], []]
    for _ in range(S):
        m = jnp.max(w, axis=0, keepdims=True)
        i = jnp.min(jnp.where(w == m, iota_n, N), axis=0, keepdims=True)
        rows_i.append(i)
        msk = (iota_n == i)
        w = jnp.where(msk, neg, w)
        mf = msk.astype(jnp.float32)
        for c in range(3):
            vrows[c].append(jnp.sum(mf * vt[c], axis=0, keepdims=True))
    idx_t = jnp.concatenate(rows_i, axis=0)
    row0 = pl.program_id(0) * BB
    rows = row0 + lax.broadcasted_iota(jnp.int32, (S, BB), 1)
    fidx_ref[...] = rows * N + idx_t
    for c in range(3):
        vnew_ref[c] = jnp.concatenate(vrows[c], axis=0)


def _tc_score(F0, vertices0, W1, b1, W2, b2):
    grid = (B // BB,)
    return pl.pallas_call(
        _tc_body,
        grid=grid,
        in_specs=[
            pl.BlockSpec((BB, N, D), lambda i: (i, 0, 0)),
            pl.BlockSpec((3, N, BB), lambda i: (0, 0, i)),
            pl.BlockSpec((D, H), lambda i: (0, 0)),
            pl.BlockSpec((1, H), lambda i: (0, 0)),
            pl.BlockSpec((H, N), lambda i: (0, 0)),
            pl.BlockSpec((1, N), lambda i: (0, 0)),
        ],
        out_specs=[
            pl.BlockSpec((BB, N, N), lambda i: (i, 0, 0)),
            pl.BlockSpec((S, BB), lambda i: (0, i)),
            pl.BlockSpec((3, S, BB), lambda i: (0, 0, i)),
        ],
        out_shape=[
            jax.ShapeDtypeStruct((B, N, N), jnp.float32),
            jax.ShapeDtypeStruct((S, B), jnp.int32),
            jax.ShapeDtypeStruct((3, S, B), jnp.float32),
        ],
        compiler_params=pltpu.CompilerParams(
            dimension_semantics=("arbitrary",),
        ),
    )(F0, vertices0, W1, b1, W2, b2)


def _sc_gather(Fflat, idxt):
    mesh = plsc.VectorSubcoreMesh(core_axis_name="c", subcore_axis_name="s")

    @functools.partial(
        pl.kernel,
        mesh=mesh,
        out_type=jax.ShapeDtypeStruct((S * B, D), jnp.float32),
        scratch_types=[
            pltpu.VMEM((S, 2 * CW), jnp.int32),
            pltpu.VMEM((3, CW, D), jnp.float32),
            pltpu.SemaphoreType.DMA,
            pltpu.SemaphoreType.DMA,
        ],
    )
    def k(f_hbm, idx_hbm, outf_hbm, idx_v, fbuf, gsem, osem):
        wid = lax.axis_index("s") * 2 + lax.axis_index("c")
        pltpu.sync_copy(idx_hbm.at[:, pl.ds((wid // 2) * 2 * CW, 2 * CW)],
                        idx_v)
        half = (wid % 2) * CW
        gs, ws = [], []
        for s in range(2):
            gs.append(pltpu.async_copy(
                f_hbm.at[idx_v.at[s, pl.ds(half, CW)]], fbuf.at[s], gsem))
        for s in range(S):
            if s >= 1 and s + 2 < S:
                ws[s - 1].wait()
            if s + 2 < S:
                gs.append(pltpu.async_copy(
                    f_hbm.at[idx_v.at[s + 2, pl.ds(half, CW)]],
                    fbuf.at[(s + 2) % 3], gsem))
            gs[s].wait()
            ws.append(pltpu.async_copy(
                fbuf.at[s % 3],
                outf_hbm.at[pl.ds(s * B + wid * CW, CW)], osem))
        ws[S - 2].wait()
        ws[S - 1].wait()

    return k(Fflat, idxt)


def kernel(F0, vertices0, k, W1, b1, W2, b2):
    score, fidx_t, vnew_t = _tc_score(
        F0, jnp.transpose(vertices0, (2, 1, 0)), W1, b1.reshape(1, H),
        W2, b2.reshape(1, N))
    Ff = _sc_gather(F0.reshape(B * N, D), fidx_t)
    F_new = jnp.transpose(Ff.reshape(S, B, D), (1, 0, 2))
    vertices_new = jnp.transpose(vnew_t, (2, 1, 0))
    return (F_new, score, vertices_new)


# speedup vs baseline: 3.2128x; 1.0020x over previous
import functools

import jax
import jax.numpy as jnp
from jax import lax
from jax.experimental import pallas as pl
from jax.experimental.pallas import tpu as pltpu
from jax.experimental.pallas import tpu_sc as plsc

B, N, S, D = 2048, 40, 20, 512
H = 256
BB = 128
NW = 32
CW = B // NW


def _tc_body(f_ref, v_ref, w1_ref, b1_ref, w2_ref, b2_ref, score_ref,
             fidx_ref, vnew_ref):
    X = f_ref[...]
    Xf = X.reshape(BB * N, D)
    h = jnp.dot(Xf, w1_ref[...], preferred_element_type=jnp.float32)
    h = h + b1_ref[...]
    h = jnp.where(h >= 0, h, 0.2 * h)
    sc = jnp.dot(h, w2_ref[...], preferred_element_type=jnp.float32)
    sc = sc + b2_ref[...]
    sc3 = sc.reshape(BB, N, N)
    score_ref[...] = sc3

    pooled = jnp.sum(X, axis=1) * (1.0 / N)
    hp = jnp.dot(pooled, w1_ref[...], preferred_element_type=jnp.float32)
    hp = hp + b1_ref[...]
    hp = jnp.where(hp >= 0, hp, 0.2 * hp)
    pli = jnp.dot(hp, w2_ref[...], preferred_element_type=jnp.float32)
    pli = pli + b2_ref[...]

    iota = lax.broadcasted_iota(jnp.int32, (BB, N), 1)
    pmax = jnp.max(pli, axis=-1, keepdims=True)
    pred = jnp.min(jnp.where(pli == pmax, iota, N), axis=-1, keepdims=True)
    onehot = (iota == pred).astype(jnp.float32)
    score_f = jnp.sum(sc3 * onehot[:, None, :], axis=-1)

    w = score_f.T
    vt = v_ref[...]
    iota_n = lax.broadcasted_iota(jnp.int32, (N, BB), 0)
    neg = jnp.float32(-jnp.inf)
    rows_i, vrows = [], [[], [], []]
    for _ in range(S):
        m = jnp.max(w, axis=0, keepdims=True)
        i = jnp.min(jnp.where(w == m, iota_n, N), axis=0, keepdims=True)
        rows_i.append(i)
        msk = (iota_n == i)
        w = jnp.where(msk, neg, w)
        mf = msk.astype(jnp.float32)
        for c in range(3):
            vrows[c].append(jnp.sum(mf * vt[c], axis=0, keepdims=True))
    idx_t = jnp.concatenate(rows_i, axis=0)
    row0 = pl.program_id(0) * BB
    rows = row0 + lax.broadcasted_iota(jnp.int32, (S, BB), 1)
    fidx_ref[...] = rows * N + idx_t
    for c in range(3):
        vnew_ref[c] = jnp.concatenate(vrows[c], axis=0)


def _tc_score(F0, vertices0, W1, b1, W2, b2):
    grid = (B // BB,)
    return pl.pallas_call(
        _tc_body,
        grid=grid,
        in_specs=[
            pl.BlockSpec((BB, N, D), lambda i: (i, 0, 0)),
            pl.BlockSpec((3, N, BB), lambda i: (0, 0, i)),
            pl.BlockSpec((D, H), lambda i: (0, 0)),
            pl.BlockSpec((1, H), lambda i: (0, 0)),
            pl.BlockSpec((H, N), lambda i: (0, 0)),
            pl.BlockSpec((1, N), lambda i: (0, 0)),
        ],
        out_specs=[
            pl.BlockSpec((BB, N, N), lambda i: (i, 0, 0)),
            pl.BlockSpec((S, BB), lambda i: (0, i)),
            pl.BlockSpec((3, S, BB), lambda i: (0, 0, i)),
        ],
        out_shape=[
            jax.ShapeDtypeStruct((B, N, N), jnp.float32),
            jax.ShapeDtypeStruct((S, B), jnp.int32),
            jax.ShapeDtypeStruct((3, S, B), jnp.float32),
        ],
        compiler_params=pltpu.CompilerParams(
            dimension_semantics=("arbitrary",),
        ),
    )(F0, vertices0, W1, b1, W2, b2)


def _sc_gather(Fflat, idxt):
    mesh = plsc.VectorSubcoreMesh(core_axis_name="c", subcore_axis_name="s")

    @functools.partial(
        pl.kernel,
        mesh=mesh,
        out_type=jax.ShapeDtypeStruct((S * B, D), jnp.float32),
        scratch_types=[
            pltpu.VMEM((S, 2 * CW), jnp.int32),
            pltpu.VMEM((3, CW, D), jnp.float32),
            pltpu.SemaphoreType.DMA,
            pltpu.SemaphoreType.DMA,
            pltpu.SemaphoreType.DMA,
            pltpu.SemaphoreType.DMA,
            pltpu.SemaphoreType.DMA,
            pltpu.SemaphoreType.DMA,
        ],
    )
    def k(f_hbm, idx_hbm, outf_hbm, idx_v, fbuf,
          g0, g1, g2, o0, o1, o2):
        gsem = [g0, g1, g2]
        osem = [o0, o1, o2]
        wid = lax.axis_index("s") * 2 + lax.axis_index("c")
        pltpu.sync_copy(idx_hbm.at[:, pl.ds((wid // 2) * 2 * CW, 2 * CW)],
                        idx_v)
        half = (wid % 2) * CW
        gs, ws = [], []
        for s in range(2):
            gs.append(pltpu.async_copy(
                f_hbm.at[idx_v.at[s, pl.ds(half, CW)]], fbuf.at[s],
                gsem[s]))
        for s in range(S):
            if s >= 1 and s + 2 < S:
                ws[s - 1].wait()
            if s + 2 < S:
                gs.append(pltpu.async_copy(
                    f_hbm.at[idx_v.at[s + 2, pl.ds(half, CW)]],
                    fbuf.at[(s + 2) % 3], gsem[(s + 2) % 3]))
            gs[s].wait()
            ws.append(pltpu.async_copy(
                fbuf.at[s % 3],
                outf_hbm.at[pl.ds(s * B + wid * CW, CW)], osem[s % 3]))
        ws[S - 2].wait()
        ws[S - 1].wait()

    return k(Fflat, idxt)


def kernel(F0, vertices0, k, W1, b1, W2, b2):
    score, fidx_t, vnew_t = _tc_score(
        F0, jnp.transpose(vertices0, (2, 1, 0)), W1, b1.reshape(1, H),
        W2, b2.reshape(1, N))
    Ff = _sc_gather(F0.reshape(B * N, D), fidx_t)
    F_new = jnp.transpose(Ff.reshape(S, B, D), (1, 0, 2))
    vertices_new = jnp.transpose(vnew_t, (2, 1, 0))
    return (F_new, score, vertices_new)
